# baseline (speedup 1.0000x reference)
import dataclasses
import functools

import jax
import jax.numpy as jnp
from jax import lax
from jax.experimental import pallas as pl
from jax.experimental.pallas import tpu as pltpu
from jax.experimental.pallas import tpu_sc as plsc

N = 10000
NP = 10112
E = 160000
TPS = 16
TROWS = NP // TPS
EPT = E // TPS

CH = 50
NCH = EPT // CH
CH3 = 25
NCH3 = (E // 2) // TPS // CH3

_mesh = plsc.VectorSubcoreMesh(core_axis_name="c", subcore_axis_name="s")

_cp = pltpu.CompilerParams()
if "needs_layout_passes" in pltpu.CompilerParams.__dataclass_fields__:
    _cp = dataclasses.replace(_cp, needs_layout_passes=False)


NH = 10240
THR = NH // TPS


@functools.partial(
    pl.kernel,
    out_type=[jax.ShapeDtypeStruct((NH,), jnp.float32),
              jax.ShapeDtypeStruct((NH,), jnp.float32)],
    mesh=_mesh,
    scratch_types=[
        pltpu.VMEM((EPT,), jnp.int32),
        pltpu.VMEM((NH,), jnp.float32),
        pltpu.VMEM((TPS, THR), jnp.float32),
        pltpu.VMEM((THR,), jnp.float32),
        pltpu.VMEM_SHARED((TPS, TPS, THR), jnp.float32),
    ],
    compiler_params=_cp,
)
def _sc_counts(src_hbm, dst_hbm, out_src, out_dst, idx_v, hist_v, red_v, res_v,
               stage_sh):
    c = lax.axis_index("c")
    t = lax.axis_index("s")

    @pl.loop(0, NH // 16)
    def _(i):
        hist_v[pl.ds(i * 16, 16)] = jnp.zeros((16,), jnp.float32)

    @pl.when(c == 0)
    def _():
        pltpu.sync_copy(src_hbm.at[pl.ds(t * EPT, EPT)], idx_v)

    @pl.when(c == 1)
    def _():
        pltpu.sync_copy(dst_hbm.at[pl.ds(t * EPT, EPT)], idx_v)

    ones16 = jnp.full((16,), 1.0, jnp.float32)

    @pl.loop(0, EPT // 16)
    def _(i):
        iv = idx_v[pl.ds(i * 16, 16)]
        plsc.addupdate_scatter(hist_v, [iv], ones16)

    for o in range(TPS):
        pltpu.sync_copy(hist_v.at[pl.ds(o * THR, THR)], stage_sh.at[o, t])
    plsc.subcore_barrier()
    pltpu.sync_copy(stage_sh.at[t], red_v)

    for s0 in range(0, THR, 16):
        acc16 = jnp.zeros((16,), jnp.float32)
        for r in range(TPS):
            acc16 = acc16 + red_v[r, pl.ds(s0, 16)]
        res_v[pl.ds(s0, 16)] = acc16

    @pl.when(c == 0)
    def _():
        pltpu.sync_copy(res_v, out_src.at[pl.ds(t * THR, THR)])

    @pl.when(c == 1)
    def _():
        pltpu.sync_copy(res_v, out_dst.at[pl.ds(t * THR, THR)])


NSEG = 25


def _zero_rows(buf, nrows):
    @pl.loop(0, nrows)
    def _(r):
        for k in range(8):
            buf[r, pl.ds(k * 16, 16)] = jnp.zeros((16,), jnp.float32)


def _zero_acc(acc_sh, zrow, t, nrows):
    chunk = (nrows // 8) * 8
    base = t * TROWS
    nfull = TROWS // chunk
    for z in range(nfull):
        pltpu.sync_copy(zrow.at[pl.ds(0, chunk)],
                        acc_sh.at[pl.ds(base + z * chunk, chunk)])
    rem = TROWS - nfull * chunk
    if rem:
        pltpu.sync_copy(zrow.at[pl.ds(0, rem)],
                        acc_sh.at[pl.ds(base + nfull * chunk, rem)])


def _make_seg_streamer(src2d_hbm, dst2d_hbm, acc_sh, bufs, tile_base,
                       seg, g):
    sidx_a, didx_a, sidx_b, didx_b = bufs[:4]
    rows = bufs[4:4 + g]
    isem0, isem1 = bufs[4 + g:6 + g]
    gsems = bufs[6 + g:6 + 2 * g]
    ssems = bufs[6 + 2 * g:6 + 3 * g]

    def seg_load(s, si, di, sync):
        src = src2d_hbm.at[pl.ds(tile_base + s * seg, seg)]
        dst = dst2d_hbm.at[pl.ds(tile_base + s * seg, seg)]
        if sync:
            pltpu.sync_copy(src, si)
            pltpu.sync_copy(dst, di)
            return None
        return (pltpu.async_copy(src, si, isem0),
                pltpu.async_copy(dst, di, isem1))

    def run(x_hbm):
        def process(si, di):
            @pl.loop(0, seg, step=g)
            def _(j):
                gh = [pltpu.async_copy(x_hbm.at[si.at[j + k]], rows[k],
                                       gsems[k]) for k in range(g)]
                sh = []
                for k in range(g):
                    gh[k].wait()
                    sh.append(pltpu.async_copy(rows[k], acc_sh.at[di.at[j + k]],
                                               ssems[k], add=True))
                for k in range(g):
                    sh[k].wait()

        seg_load(0, sidx_a, didx_a, sync=True)

        @pl.loop(0, NSEG - 1, step=2)
        def _(s):
            hb = seg_load(s + 1, sidx_b, didx_b, sync=False)
            process(sidx_a, didx_a)
            for h in hb:
                h.wait()
            ha = seg_load(s + 2, sidx_a, didx_a, sync=False)
            process(sidx_b, didx_b)
            for h in ha:
                h.wait()

        process(sidx_a, didx_a)

    return run


def _prop_scratch(ch, seg, g):
    return ([pltpu.VMEM((seg, ch), jnp.int32) for _ in range(4)]
            + [pltpu.VMEM((ch, 128), jnp.float32) for _ in range(g)]
            + [pltpu.SemaphoreType.DMA for _ in range(2 + 2 * g)])


SEG2 = 8
G2 = 4


@functools.partial(
    pl.kernel,
    out_type=jax.ShapeDtypeStruct((2, NP, 128), jnp.float32),
    mesh=_mesh,
    scratch_types=[pltpu.VMEM_SHARED((NP, 128), jnp.float32)]
                  + _prop_scratch(CH, SEG2, G2),
)
def _prop2(x_lo, x_hi, src2d_hbm, dst2d_hbm, out, acc_sh, *bufs):
    c = lax.axis_index("c")
    t = lax.axis_index("s")

    _zero_rows(bufs[4], CH)
    _zero_acc(acc_sh, bufs[4], t, CH)
    plsc.subcore_barrier()

    run = _make_seg_streamer(src2d_hbm, dst2d_hbm, acc_sh, bufs, t * NCH,
                             SEG2, G2)

    @pl.when(c == 0)
    def _():
        run(x_lo)

    @pl.when(c == 1)
    def _():
        run(x_hi)

    plsc.subcore_barrier()

    @pl.when(c == 0)
    def _():
        pltpu.sync_copy(acc_sh.at[pl.ds(t * TROWS, TROWS)],
                        out.at[0, pl.ds(t * TROWS, TROWS)])

    @pl.when(c == 1)
    def _():
        pltpu.sync_copy(acc_sh.at[pl.ds(t * TROWS, TROWS)],
                        out.at[1, pl.ds(t * TROWS, TROWS)])


SEG3 = 8
G3 = 8


@functools.partial(
    pl.kernel,
    out_type=jax.ShapeDtypeStruct((2, NP, 128), jnp.float32),
    mesh=_mesh,
    scratch_types=[pltpu.VMEM_SHARED((NP, 128), jnp.float32)]
                  + _prop_scratch(CH3, SEG3, G3),
)
def _prop3(xp, src2d_hbm, dst2d_hbm, out, acc_sh, *bufs):
    c = lax.axis_index("c")
    t = lax.axis_index("s")
    half_rows = (E // 2) // CH3

    _zero_rows(bufs[4], CH3)
    _zero_acc(acc_sh, bufs[4], t, CH3)
    plsc.subcore_barrier()

    run = _make_seg_streamer(src2d_hbm, dst2d_hbm, acc_sh, bufs,
                             c * half_rows + t * NCH3, SEG3, G3)
    run(xp)

    plsc.subcore_barrier()

    @pl.when(c == 0)
    def _():
        pltpu.sync_copy(acc_sh.at[pl.ds(t * TROWS, TROWS)],
                        out.at[0, pl.ds(t * TROWS, TROWS)])

    @pl.when(c == 1)
    def _():
        pltpu.sync_copy(acc_sh.at[pl.ds(t * TROWS, TROWS)],
                        out.at[1, pl.ds(t * TROWS, TROWS)])


RB = 1264


def _row_spec(w):
    return pl.BlockSpec((RB, w), lambda i: (i, 0))


def _full_spec(h, w):
    return pl.BlockSpec((h, w), lambda i: (0, 0))


def _prep_body(cs_ref, cd_ref, x_ref, ns_ref, nd_ref, lo_ref, hi_ref):
    ns = lax.rsqrt(jnp.maximum(cs_ref[...], 1.0))
    nd = lax.rsqrt(jnp.maximum(cd_ref[...], 1.0))
    ns_ref[...] = ns
    nd_ref[...] = nd
    xs = x_ref[...] * ns
    lo_ref[...] = xs[:, :128]
    hi_ref[...] = xs[:, 128:]


def _tc_prep(cs, cd, x):
    return pl.pallas_call(
        _prep_body,
        grid=(NP // RB,),
        in_specs=[_row_spec(1), _row_spec(1), _row_spec(256)],
        out_specs=[_row_spec(1), _row_spec(1), _row_spec(128), _row_spec(128)],
        out_shape=[
            jax.ShapeDtypeStruct((NP, 1), jnp.float32),
            jax.ShapeDtypeStruct((NP, 1), jnp.float32),
            jax.ShapeDtypeStruct((NP, 128), jnp.float32),
            jax.ShapeDtypeStruct((NP, 128), jnp.float32),
        ],
    )(cs, cd, x)


_stk_spec = pl.BlockSpec((2, RB, 128), lambda i: (0, i, 0))


def _l12_body(a_ref, nd_ref, ns_ref, w1_ref, b1_ref, w2_ref,
              glo_ref, ghi_ref):
    agg = jnp.concatenate([a_ref[0], a_ref[1]], axis=1)
    y1 = jnp.dot(agg * nd_ref[...], w1_ref[...],
                 preferred_element_type=jnp.float32)
    y1 = y1 + b1_ref[...]
    m = jnp.max(y1, axis=-1, keepdims=True)
    e = jnp.exp(y1 - m)
    h1 = e / jnp.sum(e, axis=-1, keepdims=True)
    g2 = jnp.dot(h1, w2_ref[...], preferred_element_type=jnp.float32)
    g2 = g2 * ns_ref[...]
    glo_ref[...] = g2[:, :128]
    ghi_ref[...] = g2[:, 128:]


def _tc_l12(a, nd, ns, W1, b1, W2):
    return pl.pallas_call(
        _l12_body,
        grid=(NP // RB,),
        in_specs=[_stk_spec, _row_spec(1), _row_spec(1),
                  _full_spec(256, 512), _full_spec(1, 512),
                  _full_spec(512, 256)],
        out_specs=[_row_spec(128), _row_spec(128)],
        out_shape=[
            jax.ShapeDtypeStruct((NP, 128), jnp.float32),
            jax.ShapeDtypeStruct((NP, 128), jnp.float32),
        ],
    )(a, nd, ns, W1, b1, W2)


def _l23_body(a_ref, nd_ref, ns_ref, b2_ref, w3_ref, gp_ref):
    agg = jnp.concatenate([a_ref[0], a_ref[1]], axis=1)
    y2 = agg * nd_ref[...] + b2_ref[...]
    h2 = jnp.maximum(y2, 0.0)
    g3 = jnp.dot(h2, w3_ref[...], preferred_element_type=jnp.float32)
    g3 = g3 * ns_ref[...]
    gp_ref[...] = jnp.concatenate([g3, jnp.zeros_like(g3)], axis=1)


def _tc_l23(a, nd, ns, b2, W3):
    return pl.pallas_call(
        _l23_body,
        grid=(NP // RB,),
        in_specs=[_stk_spec, _row_spec(1), _row_spec(1),
                  _full_spec(1, 256), _full_spec(256, 64)],
        out_specs=_row_spec(128),
        out_shape=jax.ShapeDtypeStruct((NP, 128), jnp.float32),
    )(a, nd, ns, b2, W3)


def _final_body(p_ref, nd_ref, b3_ref, out_ref):
    agg = p_ref[0][:, :64] + p_ref[1][:, :64]
    out_ref[...] = agg * nd_ref[...] + b3_ref[...]


def _tc_final(p, nd, b3):
    return pl.pallas_call(
        _final_body,
        grid=(NP // RB,),
        in_specs=[pl.BlockSpec((2, RB, 128), lambda i: (0, i, 0)),
                  _row_spec(1), _full_spec(1, 64)],
        out_specs=_row_spec(64),
        out_shape=jax.ShapeDtypeStruct((N, 64), jnp.float32),
    )(p, nd, b3)


def kernel(in_feat, edge_index, W1, b1, W2, b2, W3, b3):
    src2d = edge_index[0].reshape(E // CH, CH)
    dst2d = edge_index[1].reshape(E // CH, CH)
    src2d3 = edge_index[0].reshape(E // CH3, CH3)
    dst2d3 = edge_index[1].reshape(E // CH3, CH3)
    c_src, c_dst = _sc_counts(edge_index[0], edge_index[1])
    ns, nd, xs_lo, xs_hi = _tc_prep(c_src[:NP].reshape(NP, 1),
                                    c_dst[:NP].reshape(NP, 1), in_feat)
    a1 = _prop2(xs_lo, xs_hi, src2d, dst2d)
    g_lo, g_hi = _tc_l12(a1, nd, ns, W1, b1.reshape(1, -1), W2)
    a2 = _prop2(g_lo, g_hi, src2d, dst2d)
    g3p = _tc_l23(a2, nd, ns, b2.reshape(1, -1), W3)
    p3 = _prop3(g3p, src2d3, dst2d3)
    return _tc_final(p3, nd, b3.reshape(1, -1))


# speedup vs baseline: 1.0046x; 1.0046x over previous
import dataclasses
import functools

import jax
import jax.numpy as jnp
from jax import lax
from jax.experimental import pallas as pl
from jax.experimental.pallas import tpu as pltpu
from jax.experimental.pallas import tpu_sc as plsc

N = 10000
NP = 10112
E = 160000
TPS = 16
TROWS = NP // TPS
EPT = E // TPS

CH = 50
NCH = EPT // CH
CH3 = 25
NCH3 = (E // 2) // TPS // CH3

_mesh = plsc.VectorSubcoreMesh(core_axis_name="c", subcore_axis_name="s")

_cp = pltpu.CompilerParams()
if "needs_layout_passes" in pltpu.CompilerParams.__dataclass_fields__:
    _cp = dataclasses.replace(_cp, needs_layout_passes=False)


NH = 10240
THR = NH // TPS


@functools.partial(
    pl.kernel,
    out_type=[jax.ShapeDtypeStruct((NH,), jnp.float32),
              jax.ShapeDtypeStruct((NH,), jnp.float32)],
    mesh=_mesh,
    scratch_types=[
        pltpu.VMEM((EPT,), jnp.int32),
        pltpu.VMEM((NH,), jnp.float32),
        pltpu.VMEM((TPS, THR), jnp.float32),
        pltpu.VMEM((THR,), jnp.float32),
        pltpu.VMEM_SHARED((TPS, TPS, THR), jnp.float32),
    ],
    compiler_params=_cp,
)
def _sc_counts(src_hbm, dst_hbm, out_src, out_dst, idx_v, hist_v, red_v, res_v,
               stage_sh):
    c = lax.axis_index("c")
    t = lax.axis_index("s")

    @pl.loop(0, NH // 16)
    def _(i):
        hist_v[pl.ds(i * 16, 16)] = jnp.zeros((16,), jnp.float32)

    @pl.when(c == 0)
    def _():
        pltpu.sync_copy(src_hbm.at[pl.ds(t * EPT, EPT)], idx_v)

    @pl.when(c == 1)
    def _():
        pltpu.sync_copy(dst_hbm.at[pl.ds(t * EPT, EPT)], idx_v)

    ones16 = jnp.full((16,), 1.0, jnp.float32)

    @pl.loop(0, EPT // 16)
    def _(i):
        iv = idx_v[pl.ds(i * 16, 16)]
        plsc.addupdate_scatter(hist_v, [iv], ones16)

    for o in range(TPS):
        pltpu.sync_copy(hist_v.at[pl.ds(o * THR, THR)], stage_sh.at[o, t])
    plsc.subcore_barrier()
    pltpu.sync_copy(stage_sh.at[t], red_v)

    for s0 in range(0, THR, 16):
        acc16 = jnp.zeros((16,), jnp.float32)
        for r in range(TPS):
            acc16 = acc16 + red_v[r, pl.ds(s0, 16)]
        res_v[pl.ds(s0, 16)] = acc16

    @pl.when(c == 0)
    def _():
        pltpu.sync_copy(res_v, out_src.at[pl.ds(t * THR, THR)])

    @pl.when(c == 1)
    def _():
        pltpu.sync_copy(res_v, out_dst.at[pl.ds(t * THR, THR)])


NSEG = 25


def _zero_rows(buf, nrows):
    @pl.loop(0, nrows)
    def _(r):
        for k in range(8):
            buf[r, pl.ds(k * 16, 16)] = jnp.zeros((16,), jnp.float32)


def _zero_acc(acc_sh, zrow, t, nrows, sem):
    chunk = (nrows // 8) * 8
    base = t * TROWS
    nfull = TROWS // chunk
    hs = [pltpu.async_copy(zrow.at[pl.ds(0, chunk)],
                           acc_sh.at[pl.ds(base + z * chunk, chunk)], sem)
          for z in range(nfull)]
    rem = TROWS - nfull * chunk
    if rem:
        hs.append(pltpu.async_copy(zrow.at[pl.ds(0, rem)],
                                   acc_sh.at[pl.ds(base + nfull * chunk, rem)],
                                   sem))
    for h in hs:
        h.wait()


def _make_seg_streamer(src2d_hbm, dst2d_hbm, acc_sh, bufs, tile_base,
                       seg, g):
    sidx_a, didx_a, sidx_b, didx_b = bufs[:4]
    rows = bufs[4:4 + g]
    isem0, isem1 = bufs[4 + g:6 + g]
    gsems = bufs[6 + g:6 + 2 * g]
    ssems = bufs[6 + 2 * g:6 + 3 * g]

    def seg_load(s, si, di, sync):
        src = src2d_hbm.at[pl.ds(tile_base + s * seg, seg)]
        dst = dst2d_hbm.at[pl.ds(tile_base + s * seg, seg)]
        if sync:
            pltpu.sync_copy(src, si)
            pltpu.sync_copy(dst, di)
            return None
        return (pltpu.async_copy(src, si, isem0),
                pltpu.async_copy(dst, di, isem1))

    def run(x_hbm):
        def process(si, di):
            @pl.loop(0, seg, step=g)
            def _(j):
                gh = [pltpu.async_copy(x_hbm.at[si.at[j + k]], rows[k],
                                       gsems[k]) for k in range(g)]
                sh = []
                for k in range(g):
                    gh[k].wait()
                    sh.append(pltpu.async_copy(rows[k], acc_sh.at[di.at[j + k]],
                                               ssems[k], add=True))
                for k in range(g):
                    sh[k].wait()

        seg_load(0, sidx_a, didx_a, sync=True)

        @pl.loop(0, NSEG - 1, step=2)
        def _(s):
            hb = seg_load(s + 1, sidx_b, didx_b, sync=False)
            process(sidx_a, didx_a)
            for h in hb:
                h.wait()
            ha = seg_load(s + 2, sidx_a, didx_a, sync=False)
            process(sidx_b, didx_b)
            for h in ha:
                h.wait()

        process(sidx_a, didx_a)

    return run


def _prop_scratch(ch, seg, g):
    return ([pltpu.VMEM((seg, ch), jnp.int32) for _ in range(4)]
            + [pltpu.VMEM((ch, 128), jnp.float32) for _ in range(g)]
            + [pltpu.SemaphoreType.DMA for _ in range(2 + 2 * g)])


SEG2 = 8
G2 = 4


@functools.partial(
    pl.kernel,
    out_type=jax.ShapeDtypeStruct((2, NP, 128), jnp.float32),
    mesh=_mesh,
    scratch_types=[pltpu.VMEM_SHARED((NP, 128), jnp.float32)]
                  + _prop_scratch(CH, SEG2, G2),
)
def _prop2(x_lo, x_hi, src2d_hbm, dst2d_hbm, out, acc_sh, *bufs):
    c = lax.axis_index("c")
    t = lax.axis_index("s")

    _zero_rows(bufs[4], CH)
    _zero_acc(acc_sh, bufs[4], t, CH, bufs[4 + G2])
    plsc.subcore_barrier()

    run = _make_seg_streamer(src2d_hbm, dst2d_hbm, acc_sh, bufs, t * NCH,
                             SEG2, G2)

    @pl.when(c == 0)
    def _():
        run(x_lo)

    @pl.when(c == 1)
    def _():
        run(x_hi)

    plsc.subcore_barrier()

    @pl.when(c == 0)
    def _():
        pltpu.sync_copy(acc_sh.at[pl.ds(t * TROWS, TROWS)],
                        out.at[0, pl.ds(t * TROWS, TROWS)])

    @pl.when(c == 1)
    def _():
        pltpu.sync_copy(acc_sh.at[pl.ds(t * TROWS, TROWS)],
                        out.at[1, pl.ds(t * TROWS, TROWS)])


SEG3 = 8
G3 = 8


@functools.partial(
    pl.kernel,
    out_type=jax.ShapeDtypeStruct((2, NP, 128), jnp.float32),
    mesh=_mesh,
    scratch_types=[pltpu.VMEM_SHARED((NP, 128), jnp.float32)]
                  + _prop_scratch(CH3, SEG3, G3),
)
def _prop3(xp, src2d_hbm, dst2d_hbm, out, acc_sh, *bufs):
    c = lax.axis_index("c")
    t = lax.axis_index("s")
    half_rows = (E // 2) // CH3

    _zero_rows(bufs[4], CH3)
    _zero_acc(acc_sh, bufs[4], t, CH3, bufs[4 + G3])
    plsc.subcore_barrier()

    run = _make_seg_streamer(src2d_hbm, dst2d_hbm, acc_sh, bufs,
                             c * half_rows + t * NCH3, SEG3, G3)
    run(xp)

    plsc.subcore_barrier()

    @pl.when(c == 0)
    def _():
        pltpu.sync_copy(acc_sh.at[pl.ds(t * TROWS, TROWS)],
                        out.at[0, pl.ds(t * TROWS, TROWS)])

    @pl.when(c == 1)
    def _():
        pltpu.sync_copy(acc_sh.at[pl.ds(t * TROWS, TROWS)],
                        out.at[1, pl.ds(t * TROWS, TROWS)])


RB = 1264


def _row_spec(w):
    return pl.BlockSpec((RB, w), lambda i: (i, 0))


def _full_spec(h, w):
    return pl.BlockSpec((h, w), lambda i: (0, 0))


def _prep_body(cs_ref, cd_ref, x_ref, ns_ref, nd_ref, lo_ref, hi_ref):
    ns = lax.rsqrt(jnp.maximum(cs_ref[...], 1.0))
    nd = lax.rsqrt(jnp.maximum(cd_ref[...], 1.0))
    ns_ref[...] = ns
    nd_ref[...] = nd
    xs = x_ref[...] * ns
    lo_ref[...] = xs[:, :128]
    hi_ref[...] = xs[:, 128:]


def _tc_prep(cs, cd, x):
    return pl.pallas_call(
        _prep_body,
        grid=(NP // RB,),
        in_specs=[_row_spec(1), _row_spec(1), _row_spec(256)],
        out_specs=[_row_spec(1), _row_spec(1), _row_spec(128), _row_spec(128)],
        out_shape=[
            jax.ShapeDtypeStruct((NP, 1), jnp.float32),
            jax.ShapeDtypeStruct((NP, 1), jnp.float32),
            jax.ShapeDtypeStruct((NP, 128), jnp.float32),
            jax.ShapeDtypeStruct((NP, 128), jnp.float32),
        ],
    )(cs, cd, x)


_stk_spec = pl.BlockSpec((2, RB, 128), lambda i: (0, i, 0))


def _l12_body(a_ref, nd_ref, ns_ref, w1_ref, b1_ref, w2_ref,
              glo_ref, ghi_ref):
    agg = jnp.concatenate([a_ref[0], a_ref[1]], axis=1)
    y1 = jnp.dot(agg * nd_ref[...], w1_ref[...],
                 preferred_element_type=jnp.float32)
    y1 = y1 + b1_ref[...]
    m = jnp.max(y1, axis=-1, keepdims=True)
    e = jnp.exp(y1 - m)
    h1 = e / jnp.sum(e, axis=-1, keepdims=True)
    g2 = jnp.dot(h1, w2_ref[...], preferred_element_type=jnp.float32)
    g2 = g2 * ns_ref[...]
    glo_ref[...] = g2[:, :128]
    ghi_ref[...] = g2[:, 128:]


def _tc_l12(a, nd, ns, W1, b1, W2):
    return pl.pallas_call(
        _l12_body,
        grid=(NP // RB,),
        in_specs=[_stk_spec, _row_spec(1), _row_spec(1),
                  _full_spec(256, 512), _full_spec(1, 512),
                  _full_spec(512, 256)],
        out_specs=[_row_spec(128), _row_spec(128)],
        out_shape=[
            jax.ShapeDtypeStruct((NP, 128), jnp.float32),
            jax.ShapeDtypeStruct((NP, 128), jnp.float32),
        ],
    )(a, nd, ns, W1, b1, W2)


def _l23_body(a_ref, nd_ref, ns_ref, b2_ref, w3_ref, gp_ref):
    agg = jnp.concatenate([a_ref[0], a_ref[1]], axis=1)
    y2 = agg * nd_ref[...] + b2_ref[...]
    h2 = jnp.maximum(y2, 0.0)
    g3 = jnp.dot(h2, w3_ref[...], preferred_element_type=jnp.float32)
    g3 = g3 * ns_ref[...]
    gp_ref[...] = jnp.concatenate([g3, jnp.zeros_like(g3)], axis=1)


def _tc_l23(a, nd, ns, b2, W3):
    return pl.pallas_call(
        _l23_body,
        grid=(NP // RB,),
        in_specs=[_stk_spec, _row_spec(1), _row_spec(1),
                  _full_spec(1, 256), _full_spec(256, 64)],
        out_specs=_row_spec(128),
        out_shape=jax.ShapeDtypeStruct((NP, 128), jnp.float32),
    )(a, nd, ns, b2, W3)


def _final_body(p_ref, nd_ref, b3_ref, out_ref):
    agg = p_ref[0][:, :64] + p_ref[1][:, :64]
    out_ref[...] = agg * nd_ref[...] + b3_ref[...]


def _tc_final(p, nd, b3):
    return pl.pallas_call(
        _final_body,
        grid=(NP // RB,),
        in_specs=[pl.BlockSpec((2, RB, 128), lambda i: (0, i, 0)),
                  _row_spec(1), _full_spec(1, 64)],
        out_specs=_row_spec(64),
        out_shape=jax.ShapeDtypeStruct((N, 64), jnp.float32),
    )(p, nd, b3)


def kernel(in_feat, edge_index, W1, b1, W2, b2, W3, b3):
    src2d = edge_index[0].reshape(E // CH, CH)
    dst2d = edge_index[1].reshape(E // CH, CH)
    src2d3 = edge_index[0].reshape(E // CH3, CH3)
    dst2d3 = edge_index[1].reshape(E // CH3, CH3)
    c_src, c_dst = _sc_counts(edge_index[0], edge_index[1])
    ns, nd, xs_lo, xs_hi = _tc_prep(c_src[:NP].reshape(NP, 1),
                                    c_dst[:NP].reshape(NP, 1), in_feat)
    a1 = _prop2(xs_lo, xs_hi, src2d, dst2d)
    g_lo, g_hi = _tc_l12(a1, nd, ns, W1, b1.reshape(1, -1), W2)
    a2 = _prop2(g_lo, g_hi, src2d, dst2d)
    g3p = _tc_l23(a2, nd, ns, b2.reshape(1, -1), W3)
    p3 = _prop3(g3p, src2d3, dst2d3)
    return _tc_final(p3, nd, b3.reshape(1, -1))


# speedup vs baseline: 1.0955x; 1.0905x over previous
import dataclasses
import functools

import jax
import jax.numpy as jnp
from jax import lax
from jax.experimental import pallas as pl
from jax.experimental.pallas import tpu as pltpu
from jax.experimental.pallas import tpu_sc as plsc

N = 10000
NP = 10112
E = 160000
TPS = 16
TROWS = NP // TPS
EPT = E // TPS

CH = 50
NCH = EPT // CH
CH3 = 25
NCH3 = (E // 2) // TPS // CH3

_mesh = plsc.VectorSubcoreMesh(core_axis_name="c", subcore_axis_name="s")

_cp = pltpu.CompilerParams()
if "needs_layout_passes" in pltpu.CompilerParams.__dataclass_fields__:
    _cp = dataclasses.replace(_cp, needs_layout_passes=False)


NH = 10240
THR = NH // TPS


@functools.partial(
    pl.kernel,
    out_type=[jax.ShapeDtypeStruct((NH,), jnp.float32),
              jax.ShapeDtypeStruct((NH,), jnp.float32)],
    mesh=_mesh,
    scratch_types=[
        pltpu.VMEM((EPT,), jnp.int32),
        pltpu.VMEM((NH,), jnp.float32),
        pltpu.VMEM((TPS, THR), jnp.float32),
        pltpu.VMEM((THR,), jnp.float32),
        pltpu.VMEM_SHARED((TPS, TPS, THR), jnp.float32),
    ],
    compiler_params=_cp,
)
def _sc_counts(src_hbm, dst_hbm, out_src, out_dst, idx_v, hist_v, red_v, res_v,
               stage_sh):
    c = lax.axis_index("c")
    t = lax.axis_index("s")

    @pl.loop(0, NH // 16)
    def _(i):
        hist_v[pl.ds(i * 16, 16)] = jnp.zeros((16,), jnp.float32)

    @pl.when(c == 0)
    def _():
        pltpu.sync_copy(src_hbm.at[pl.ds(t * EPT, EPT)], idx_v)

    @pl.when(c == 1)
    def _():
        pltpu.sync_copy(dst_hbm.at[pl.ds(t * EPT, EPT)], idx_v)

    ones16 = jnp.full((16,), 1.0, jnp.float32)

    @pl.loop(0, EPT // 16)
    def _(i):
        iv = idx_v[pl.ds(i * 16, 16)]
        plsc.addupdate_scatter(hist_v, [iv], ones16)

    for o in range(TPS):
        pltpu.sync_copy(hist_v.at[pl.ds(o * THR, THR)], stage_sh.at[o, t])
    plsc.subcore_barrier()
    pltpu.sync_copy(stage_sh.at[t], red_v)

    for s0 in range(0, THR, 16):
        acc16 = jnp.zeros((16,), jnp.float32)
        for r in range(TPS):
            acc16 = acc16 + red_v[r, pl.ds(s0, 16)]
        res_v[pl.ds(s0, 16)] = acc16

    @pl.when(c == 0)
    def _():
        pltpu.sync_copy(res_v, out_src.at[pl.ds(t * THR, THR)])

    @pl.when(c == 1)
    def _():
        pltpu.sync_copy(res_v, out_dst.at[pl.ds(t * THR, THR)])


NSEG = 25


def _zero_rows(buf, nrows):
    @pl.loop(0, nrows)
    def _(r):
        for k in range(8):
            buf[r, pl.ds(k * 16, 16)] = jnp.zeros((16,), jnp.float32)


def _zero_acc(acc_sh, zrow, t, nrows, sem):
    chunk = (nrows // 8) * 8
    base = t * TROWS
    nfull = TROWS // chunk
    hs = [pltpu.async_copy(zrow.at[pl.ds(0, chunk)],
                           acc_sh.at[pl.ds(base + z * chunk, chunk)], sem)
          for z in range(nfull)]
    rem = TROWS - nfull * chunk
    if rem:
        hs.append(pltpu.async_copy(zrow.at[pl.ds(0, rem)],
                                   acc_sh.at[pl.ds(base + nfull * chunk, rem)],
                                   sem))
    for h in hs:
        h.wait()


def _make_seg_streamer(src2d_hbm, dst2d_hbm, acc_sh, bufs, tile_base,
                       seg, g):
    sidx_a, didx_a, sidx_b, didx_b = bufs[:4]
    rows = bufs[4:4 + g]
    isem0, isem1 = bufs[4 + g:6 + g]
    gsems = bufs[6 + g:6 + 2 * g]
    ssems = bufs[6 + 2 * g:6 + 3 * g]

    def seg_load(s, si, di, sync):
        src = src2d_hbm.at[pl.ds(tile_base + s * seg, seg)]
        dst = dst2d_hbm.at[pl.ds(tile_base + s * seg, seg)]
        if sync:
            pltpu.sync_copy(src, si)
            pltpu.sync_copy(dst, di)
            return None
        return (pltpu.async_copy(src, si, isem0),
                pltpu.async_copy(dst, di, isem1))

    def run(x_hbm):
        def process(si, di):
            gh = [None] * seg
            sh = [None] * seg
            for j in range(seg):
                if j >= g:
                    sh[j - g].wait()
                gh[j] = pltpu.async_copy(x_hbm.at[si.at[j]], rows[j % g],
                                         gsems[j % g])
                k = j - (g - 1)
                if k >= 0:
                    gh[k].wait()
                    sh[k] = pltpu.async_copy(rows[k % g], acc_sh.at[di.at[k]],
                                             ssems[k % g], add=True)
            for k in range(max(seg - g + 1, 0), seg):
                gh[k].wait()
                sh[k] = pltpu.async_copy(rows[k % g], acc_sh.at[di.at[k]],
                                         ssems[k % g], add=True)
            for k in range(max(seg - g, 0), seg):
                sh[k].wait()

        seg_load(0, sidx_a, didx_a, sync=True)

        @pl.loop(0, NSEG - 1, step=2)
        def _(s):
            hb = seg_load(s + 1, sidx_b, didx_b, sync=False)
            process(sidx_a, didx_a)
            for h in hb:
                h.wait()
            ha = seg_load(s + 2, sidx_a, didx_a, sync=False)
            process(sidx_b, didx_b)
            for h in ha:
                h.wait()

        process(sidx_a, didx_a)

    return run


def _prop_scratch(ch, seg, g):
    return ([pltpu.VMEM((seg, ch), jnp.int32) for _ in range(4)]
            + [pltpu.VMEM((ch, 128), jnp.float32) for _ in range(g)]
            + [pltpu.SemaphoreType.DMA for _ in range(2 + 2 * g)])


SEG2 = 8
G2 = 5


@functools.partial(
    pl.kernel,
    out_type=jax.ShapeDtypeStruct((2, NP, 128), jnp.float32),
    mesh=_mesh,
    scratch_types=[pltpu.VMEM_SHARED((NP, 128), jnp.float32)]
                  + _prop_scratch(CH, SEG2, G2),
)
def _prop2(x_lo, x_hi, src2d_hbm, dst2d_hbm, out, acc_sh, *bufs):
    c = lax.axis_index("c")
    t = lax.axis_index("s")

    _zero_rows(bufs[4], CH)
    _zero_acc(acc_sh, bufs[4], t, CH, bufs[4 + G2])
    plsc.subcore_barrier()

    run = _make_seg_streamer(src2d_hbm, dst2d_hbm, acc_sh, bufs, t * NCH,
                             SEG2, G2)

    @pl.when(c == 0)
    def _():
        run(x_lo)

    @pl.when(c == 1)
    def _():
        run(x_hi)

    plsc.subcore_barrier()

    @pl.when(c == 0)
    def _():
        pltpu.sync_copy(acc_sh.at[pl.ds(t * TROWS, TROWS)],
                        out.at[0, pl.ds(t * TROWS, TROWS)])

    @pl.when(c == 1)
    def _():
        pltpu.sync_copy(acc_sh.at[pl.ds(t * TROWS, TROWS)],
                        out.at[1, pl.ds(t * TROWS, TROWS)])


SEG3 = 8
G3 = 8


@functools.partial(
    pl.kernel,
    out_type=jax.ShapeDtypeStruct((2, NP, 128), jnp.float32),
    mesh=_mesh,
    scratch_types=[pltpu.VMEM_SHARED((NP, 128), jnp.float32)]
                  + _prop_scratch(CH3, SEG3, G3),
)
def _prop3(xp, src2d_hbm, dst2d_hbm, out, acc_sh, *bufs):
    c = lax.axis_index("c")
    t = lax.axis_index("s")
    half_rows = (E // 2) // CH3

    _zero_rows(bufs[4], CH3)
    _zero_acc(acc_sh, bufs[4], t, CH3, bufs[4 + G3])
    plsc.subcore_barrier()

    run = _make_seg_streamer(src2d_hbm, dst2d_hbm, acc_sh, bufs,
                             c * half_rows + t * NCH3, SEG3, G3)
    run(xp)

    plsc.subcore_barrier()

    @pl.when(c == 0)
    def _():
        pltpu.sync_copy(acc_sh.at[pl.ds(t * TROWS, TROWS)],
                        out.at[0, pl.ds(t * TROWS, TROWS)])

    @pl.when(c == 1)
    def _():
        pltpu.sync_copy(acc_sh.at[pl.ds(t * TROWS, TROWS)],
                        out.at[1, pl.ds(t * TROWS, TROWS)])


RB = 1264


def _row_spec(w):
    return pl.BlockSpec((RB, w), lambda i: (i, 0))


def _full_spec(h, w):
    return pl.BlockSpec((h, w), lambda i: (0, 0))


def _prep_body(cs_ref, cd_ref, x_ref, ns_ref, nd_ref, lo_ref, hi_ref):
    ns = lax.rsqrt(jnp.maximum(cs_ref[...], 1.0))
    nd = lax.rsqrt(jnp.maximum(cd_ref[...], 1.0))
    ns_ref[...] = ns
    nd_ref[...] = nd
    xs = x_ref[...] * ns
    lo_ref[...] = xs[:, :128]
    hi_ref[...] = xs[:, 128:]


def _tc_prep(cs, cd, x):
    return pl.pallas_call(
        _prep_body,
        grid=(NP // RB,),
        in_specs=[_row_spec(1), _row_spec(1), _row_spec(256)],
        out_specs=[_row_spec(1), _row_spec(1), _row_spec(128), _row_spec(128)],
        out_shape=[
            jax.ShapeDtypeStruct((NP, 1), jnp.float32),
            jax.ShapeDtypeStruct((NP, 1), jnp.float32),
            jax.ShapeDtypeStruct((NP, 128), jnp.float32),
            jax.ShapeDtypeStruct((NP, 128), jnp.float32),
        ],
    )(cs, cd, x)


_stk_spec = pl.BlockSpec((2, RB, 128), lambda i: (0, i, 0))


def _l12_body(a_ref, nd_ref, ns_ref, w1_ref, b1_ref, w2_ref,
              glo_ref, ghi_ref):
    agg = jnp.concatenate([a_ref[0], a_ref[1]], axis=1)
    y1 = jnp.dot(agg * nd_ref[...], w1_ref[...],
                 preferred_element_type=jnp.float32)
    y1 = y1 + b1_ref[...]
    m = jnp.max(y1, axis=-1, keepdims=True)
    e = jnp.exp(y1 - m)
    h1 = e / jnp.sum(e, axis=-1, keepdims=True)
    g2 = jnp.dot(h1, w2_ref[...], preferred_element_type=jnp.float32)
    g2 = g2 * ns_ref[...]
    glo_ref[...] = g2[:, :128]
    ghi_ref[...] = g2[:, 128:]


def _tc_l12(a, nd, ns, W1, b1, W2):
    return pl.pallas_call(
        _l12_body,
        grid=(NP // RB,),
        in_specs=[_stk_spec, _row_spec(1), _row_spec(1),
                  _full_spec(256, 512), _full_spec(1, 512),
                  _full_spec(512, 256)],
        out_specs=[_row_spec(128), _row_spec(128)],
        out_shape=[
            jax.ShapeDtypeStruct((NP, 128), jnp.float32),
            jax.ShapeDtypeStruct((NP, 128), jnp.float32),
        ],
    )(a, nd, ns, W1, b1, W2)


def _l23_body(a_ref, nd_ref, ns_ref, b2_ref, w3_ref, gp_ref):
    agg = jnp.concatenate([a_ref[0], a_ref[1]], axis=1)
    y2 = agg * nd_ref[...] + b2_ref[...]
    h2 = jnp.maximum(y2, 0.0)
    g3 = jnp.dot(h2, w3_ref[...], preferred_element_type=jnp.float32)
    g3 = g3 * ns_ref[...]
    gp_ref[...] = jnp.concatenate([g3, jnp.zeros_like(g3)], axis=1)


def _tc_l23(a, nd, ns, b2, W3):
    return pl.pallas_call(
        _l23_body,
        grid=(NP // RB,),
        in_specs=[_stk_spec, _row_spec(1), _row_spec(1),
                  _full_spec(1, 256), _full_spec(256, 64)],
        out_specs=_row_spec(128),
        out_shape=jax.ShapeDtypeStruct((NP, 128), jnp.float32),
    )(a, nd, ns, b2, W3)


def _final_body(p_ref, nd_ref, b3_ref, out_ref):
    agg = p_ref[0][:, :64] + p_ref[1][:, :64]
    out_ref[...] = agg * nd_ref[...] + b3_ref[...]


def _tc_final(p, nd, b3):
    return pl.pallas_call(
        _final_body,
        grid=(NP // RB,),
        in_specs=[pl.BlockSpec((2, RB, 128), lambda i: (0, i, 0)),
                  _row_spec(1), _full_spec(1, 64)],
        out_specs=_row_spec(64),
        out_shape=jax.ShapeDtypeStruct((N, 64), jnp.float32),
    )(p, nd, b3)


def kernel(in_feat, edge_index, W1, b1, W2, b2, W3, b3):
    src2d = edge_index[0].reshape(E // CH, CH)
    dst2d = edge_index[1].reshape(E // CH, CH)
    src2d3 = edge_index[0].reshape(E // CH3, CH3)
    dst2d3 = edge_index[1].reshape(E // CH3, CH3)
    c_src, c_dst = _sc_counts(edge_index[0], edge_index[1])
    ns, nd, xs_lo, xs_hi = _tc_prep(c_src[:NP].reshape(NP, 1),
                                    c_dst[:NP].reshape(NP, 1), in_feat)
    a1 = _prop2(xs_lo, xs_hi, src2d, dst2d)
    g_lo, g_hi = _tc_l12(a1, nd, ns, W1, b1.reshape(1, -1), W2)
    a2 = _prop2(g_lo, g_hi, src2d, dst2d)
    g3p = _tc_l23(a2, nd, ns, b2.reshape(1, -1), W3)
    p3 = _prop3(g3p, src2d3, dst2d3)
    return _tc_final(p3, nd, b3.reshape(1, -1))


# speedup vs baseline: 1.2345x; 1.1268x over previous
import dataclasses
import functools

import jax
import jax.numpy as jnp
from jax import lax
from jax.experimental import pallas as pl
from jax.experimental.pallas import tpu as pltpu
from jax.experimental.pallas import tpu_sc as plsc

N = 10000
NP = 10112
E = 160000
TPS = 16
TROWS = NP // TPS
EPT = E // TPS

CH = 50
NCH = EPT // CH
CH3 = 25
NCH3 = (E // 2) // TPS // CH3

_mesh = plsc.VectorSubcoreMesh(core_axis_name="c", subcore_axis_name="s")

_cp = pltpu.CompilerParams()
if "needs_layout_passes" in pltpu.CompilerParams.__dataclass_fields__:
    _cp = dataclasses.replace(_cp, needs_layout_passes=False)


NH = 10240
THR = NH // TPS


@functools.partial(
    pl.kernel,
    out_type=[jax.ShapeDtypeStruct((NH,), jnp.float32),
              jax.ShapeDtypeStruct((NH,), jnp.float32)],
    mesh=_mesh,
    scratch_types=[
        pltpu.VMEM((EPT,), jnp.int32),
        pltpu.VMEM((NH,), jnp.float32),
        pltpu.VMEM((TPS, THR), jnp.float32),
        pltpu.VMEM((THR,), jnp.float32),
        pltpu.VMEM_SHARED((TPS, TPS, THR), jnp.float32),
    ],
    compiler_params=_cp,
)
def _sc_counts(src_hbm, dst_hbm, out_src, out_dst, idx_v, hist_v, red_v, res_v,
               stage_sh):
    c = lax.axis_index("c")
    t = lax.axis_index("s")

    @pl.loop(0, NH // 16)
    def _(i):
        hist_v[pl.ds(i * 16, 16)] = jnp.zeros((16,), jnp.float32)

    @pl.when(c == 0)
    def _():
        pltpu.sync_copy(src_hbm.at[pl.ds(t * EPT, EPT)], idx_v)

    @pl.when(c == 1)
    def _():
        pltpu.sync_copy(dst_hbm.at[pl.ds(t * EPT, EPT)], idx_v)

    ones16 = jnp.full((16,), 1.0, jnp.float32)

    @pl.loop(0, EPT // 16)
    def _(i):
        iv = idx_v[pl.ds(i * 16, 16)]
        plsc.addupdate_scatter(hist_v, [iv], ones16)

    for o in range(TPS):
        pltpu.sync_copy(hist_v.at[pl.ds(o * THR, THR)], stage_sh.at[o, t])
    plsc.subcore_barrier()
    pltpu.sync_copy(stage_sh.at[t], red_v)

    for s0 in range(0, THR, 16):
        acc16 = jnp.zeros((16,), jnp.float32)
        for r in range(TPS):
            acc16 = acc16 + red_v[r, pl.ds(s0, 16)]
        res_v[pl.ds(s0, 16)] = acc16

    @pl.when(c == 0)
    def _():
        pltpu.sync_copy(res_v, out_src.at[pl.ds(t * THR, THR)])

    @pl.when(c == 1)
    def _():
        pltpu.sync_copy(res_v, out_dst.at[pl.ds(t * THR, THR)])


NSEG = 25


def _zero_rows(buf, nrows):
    @pl.loop(0, nrows)
    def _(r):
        for k in range(8):
            buf[r, pl.ds(k * 16, 16)] = jnp.zeros((16,), jnp.float32)


def _zero_acc(acc_sh, zrow, t, nrows, sem):
    chunk = (nrows // 8) * 8
    base = t * TROWS
    nfull = TROWS // chunk
    hs = [pltpu.async_copy(zrow.at[pl.ds(0, chunk)],
                           acc_sh.at[pl.ds(base + z * chunk, chunk)], sem)
          for z in range(nfull)]
    rem = TROWS - nfull * chunk
    if rem:
        hs.append(pltpu.async_copy(zrow.at[pl.ds(0, rem)],
                                   acc_sh.at[pl.ds(base + nfull * chunk, rem)],
                                   sem))
    for h in hs:
        h.wait()


def _make_seg_streamer(src2d_hbm, dst2d_hbm, acc_sh, bufs, tile_base,
                       seg, g):
    sidx_a, didx_a, sidx_b, didx_b = bufs[:4]
    rows = bufs[4:4 + g]
    isem0, isem1 = bufs[4 + g:6 + g]
    gsems = bufs[6 + g:6 + 2 * g]
    ssems = bufs[6 + 2 * g:6 + 3 * g]

    def seg_load(s, si, di, sync):
        src = src2d_hbm.at[pl.ds(tile_base + s * seg, seg)]
        dst = dst2d_hbm.at[pl.ds(tile_base + s * seg, seg)]
        if sync:
            pltpu.sync_copy(src, si)
            pltpu.sync_copy(dst, di)
            return None
        return (pltpu.async_copy(src, si, isem0),
                pltpu.async_copy(dst, di, isem1))

    def run(x_hbm):
        def sweep(segments, hooks=None):
            total = seg * len(segments)
            gh = [None] * total
            sh = [None] * total
            late = []

            def fire_s(k):
                di = segments[k // seg][1]
                gh[k].wait()
                sh[k] = pltpu.async_copy(rows[k % g], acc_sh.at[di.at[k % seg]],
                                         ssems[k % g], add=True)

            for j in range(total):
                if hooks and j in hooks:
                    late.extend(hooks[j]() or ())
                if j >= g:
                    sh[j - g].wait()
                si = segments[j // seg][0]
                gh[j] = pltpu.async_copy(x_hbm.at[si.at[j % seg]], rows[j % g],
                                         gsems[j % g])
                if j - (g - 1) >= 0:
                    fire_s(j - (g - 1))
            if hooks and total in hooks:
                late.extend(hooks[total]() or ())
            for k in range(max(total - g + 1, 0), total):
                fire_s(k)
            for k in range(max(total - g, 0), total):
                sh[k].wait()
            for h in late:
                h.wait()

        seg_load(0, sidx_a, didx_a, sync=True)

        @pl.loop(0, NSEG - 1, step=2)
        def _(s):
            hb = seg_load(s + 1, sidx_b, didx_b, sync=False)

            def wait_hb():
                for h in hb:
                    h.wait()

            def fire_ha():
                return seg_load(s + 2, sidx_a, didx_a, sync=False)

            sweep([(sidx_a, didx_a), (sidx_b, didx_b)],
                  {seg: wait_hb, min(seg + g, 2 * seg): fire_ha})

        sweep([(sidx_a, didx_a)])

    return run


def _prop_scratch(ch, seg, g):
    return ([pltpu.VMEM((seg, ch), jnp.int32) for _ in range(4)]
            + [pltpu.VMEM((ch, 128), jnp.float32) for _ in range(g)]
            + [pltpu.SemaphoreType.DMA for _ in range(2 + 2 * g)])


SEG2 = 8
G2 = 5


@functools.partial(
    pl.kernel,
    out_type=jax.ShapeDtypeStruct((2, NP, 128), jnp.float32),
    mesh=_mesh,
    scratch_types=[pltpu.VMEM_SHARED((NP, 128), jnp.float32)]
                  + _prop_scratch(CH, SEG2, G2),
)
def _prop2(x_lo, x_hi, src2d_hbm, dst2d_hbm, out, acc_sh, *bufs):
    c = lax.axis_index("c")
    t = lax.axis_index("s")

    _zero_rows(bufs[4], CH)
    _zero_acc(acc_sh, bufs[4], t, CH, bufs[4 + G2])
    plsc.subcore_barrier()

    run = _make_seg_streamer(src2d_hbm, dst2d_hbm, acc_sh, bufs, t * NCH,
                             SEG2, G2)

    @pl.when(c == 0)
    def _():
        run(x_lo)

    @pl.when(c == 1)
    def _():
        run(x_hi)

    plsc.subcore_barrier()

    @pl.when(c == 0)
    def _():
        pltpu.sync_copy(acc_sh.at[pl.ds(t * TROWS, TROWS)],
                        out.at[0, pl.ds(t * TROWS, TROWS)])

    @pl.when(c == 1)
    def _():
        pltpu.sync_copy(acc_sh.at[pl.ds(t * TROWS, TROWS)],
                        out.at[1, pl.ds(t * TROWS, TROWS)])


SEG3 = 8
G3 = 8


@functools.partial(
    pl.kernel,
    out_type=jax.ShapeDtypeStruct((2, NP, 128), jnp.float32),
    mesh=_mesh,
    scratch_types=[pltpu.VMEM_SHARED((NP, 128), jnp.float32)]
                  + _prop_scratch(CH3, SEG3, G3),
)
def _prop3(xp, src2d_hbm, dst2d_hbm, out, acc_sh, *bufs):
    c = lax.axis_index("c")
    t = lax.axis_index("s")
    half_rows = (E // 2) // CH3

    _zero_rows(bufs[4], CH3)
    _zero_acc(acc_sh, bufs[4], t, CH3, bufs[4 + G3])
    plsc.subcore_barrier()

    run = _make_seg_streamer(src2d_hbm, dst2d_hbm, acc_sh, bufs,
                             c * half_rows + t * NCH3, SEG3, G3)
    run(xp)

    plsc.subcore_barrier()

    @pl.when(c == 0)
    def _():
        pltpu.sync_copy(acc_sh.at[pl.ds(t * TROWS, TROWS)],
                        out.at[0, pl.ds(t * TROWS, TROWS)])

    @pl.when(c == 1)
    def _():
        pltpu.sync_copy(acc_sh.at[pl.ds(t * TROWS, TROWS)],
                        out.at[1, pl.ds(t * TROWS, TROWS)])


RB = 1264


def _row_spec(w):
    return pl.BlockSpec((RB, w), lambda i: (i, 0))


def _full_spec(h, w):
    return pl.BlockSpec((h, w), lambda i: (0, 0))


def _prep_body(cs_ref, cd_ref, x_ref, ns_ref, nd_ref, lo_ref, hi_ref):
    ns = lax.rsqrt(jnp.maximum(cs_ref[...], 1.0))
    nd = lax.rsqrt(jnp.maximum(cd_ref[...], 1.0))
    ns_ref[...] = ns
    nd_ref[...] = nd
    xs = x_ref[...] * ns
    lo_ref[...] = xs[:, :128]
    hi_ref[...] = xs[:, 128:]


def _tc_prep(cs, cd, x):
    return pl.pallas_call(
        _prep_body,
        grid=(NP // RB,),
        in_specs=[_row_spec(1), _row_spec(1), _row_spec(256)],
        out_specs=[_row_spec(1), _row_spec(1), _row_spec(128), _row_spec(128)],
        out_shape=[
            jax.ShapeDtypeStruct((NP, 1), jnp.float32),
            jax.ShapeDtypeStruct((NP, 1), jnp.float32),
            jax.ShapeDtypeStruct((NP, 128), jnp.float32),
            jax.ShapeDtypeStruct((NP, 128), jnp.float32),
        ],
    )(cs, cd, x)


_stk_spec = pl.BlockSpec((2, RB, 128), lambda i: (0, i, 0))


def _l12_body(a_ref, nd_ref, ns_ref, w1_ref, b1_ref, w2_ref,
              glo_ref, ghi_ref):
    agg = jnp.concatenate([a_ref[0], a_ref[1]], axis=1)
    y1 = jnp.dot(agg * nd_ref[...], w1_ref[...],
                 preferred_element_type=jnp.float32)
    y1 = y1 + b1_ref[...]
    m = jnp.max(y1, axis=-1, keepdims=True)
    e = jnp.exp(y1 - m)
    h1 = e / jnp.sum(e, axis=-1, keepdims=True)
    g2 = jnp.dot(h1, w2_ref[...], preferred_element_type=jnp.float32)
    g2 = g2 * ns_ref[...]
    glo_ref[...] = g2[:, :128]
    ghi_ref[...] = g2[:, 128:]


def _tc_l12(a, nd, ns, W1, b1, W2):
    return pl.pallas_call(
        _l12_body,
        grid=(NP // RB,),
        in_specs=[_stk_spec, _row_spec(1), _row_spec(1),
                  _full_spec(256, 512), _full_spec(1, 512),
                  _full_spec(512, 256)],
        out_specs=[_row_spec(128), _row_spec(128)],
        out_shape=[
            jax.ShapeDtypeStruct((NP, 128), jnp.float32),
            jax.ShapeDtypeStruct((NP, 128), jnp.float32),
        ],
    )(a, nd, ns, W1, b1, W2)


def _l23_body(a_ref, nd_ref, ns_ref, b2_ref, w3_ref, gp_ref):
    agg = jnp.concatenate([a_ref[0], a_ref[1]], axis=1)
    y2 = agg * nd_ref[...] + b2_ref[...]
    h2 = jnp.maximum(y2, 0.0)
    g3 = jnp.dot(h2, w3_ref[...], preferred_element_type=jnp.float32)
    g3 = g3 * ns_ref[...]
    gp_ref[...] = jnp.concatenate([g3, jnp.zeros_like(g3)], axis=1)


def _tc_l23(a, nd, ns, b2, W3):
    return pl.pallas_call(
        _l23_body,
        grid=(NP // RB,),
        in_specs=[_stk_spec, _row_spec(1), _row_spec(1),
                  _full_spec(1, 256), _full_spec(256, 64)],
        out_specs=_row_spec(128),
        out_shape=jax.ShapeDtypeStruct((NP, 128), jnp.float32),
    )(a, nd, ns, b2, W3)


def _final_body(p_ref, nd_ref, b3_ref, out_ref):
    agg = p_ref[0][:, :64] + p_ref[1][:, :64]
    out_ref[...] = agg * nd_ref[...] + b3_ref[...]


def _tc_final(p, nd, b3):
    return pl.pallas_call(
        _final_body,
        grid=(NP // RB,),
        in_specs=[pl.BlockSpec((2, RB, 128), lambda i: (0, i, 0)),
                  _row_spec(1), _full_spec(1, 64)],
        out_specs=_row_spec(64),
        out_shape=jax.ShapeDtypeStruct((N, 64), jnp.float32),
    )(p, nd, b3)


def kernel(in_feat, edge_index, W1, b1, W2, b2, W3, b3):
    src2d = edge_index[0].reshape(E // CH, CH)
    dst2d = edge_index[1].reshape(E // CH, CH)
    src2d3 = edge_index[0].reshape(E // CH3, CH3)
    dst2d3 = edge_index[1].reshape(E // CH3, CH3)
    c_src, c_dst = _sc_counts(edge_index[0], edge_index[1])
    ns, nd, xs_lo, xs_hi = _tc_prep(c_src[:NP].reshape(NP, 1),
                                    c_dst[:NP].reshape(NP, 1), in_feat)
    a1 = _prop2(xs_lo, xs_hi, src2d, dst2d)
    g_lo, g_hi = _tc_l12(a1, nd, ns, W1, b1.reshape(1, -1), W2)
    a2 = _prop2(g_lo, g_hi, src2d, dst2d)
    g3p = _tc_l23(a2, nd, ns, b2.reshape(1, -1), W3)
    p3 = _prop3(g3p, src2d3, dst2d3)
    return _tc_final(p3, nd, b3.reshape(1, -1))


# speedup vs baseline: 1.2554x; 1.0170x over previous
import dataclasses
import functools

import jax
import jax.numpy as jnp
from jax import lax
from jax.experimental import pallas as pl
from jax.experimental.pallas import tpu as pltpu
from jax.experimental.pallas import tpu_sc as plsc

N = 10000
NP = 10112
E = 160000
TPS = 16
TROWS = NP // TPS
EPT = E // TPS

CH = 50
NCH = EPT // CH
CH3 = 25
NCH3 = (E // 2) // TPS // CH3

_mesh = plsc.VectorSubcoreMesh(core_axis_name="c", subcore_axis_name="s")

_cp = pltpu.CompilerParams()
if "needs_layout_passes" in pltpu.CompilerParams.__dataclass_fields__:
    _cp = dataclasses.replace(_cp, needs_layout_passes=False)


NH = 10240
THR = NH // TPS


@functools.partial(
    pl.kernel,
    out_type=[jax.ShapeDtypeStruct((NH,), jnp.float32),
              jax.ShapeDtypeStruct((NH,), jnp.float32)],
    mesh=_mesh,
    scratch_types=[
        pltpu.VMEM((EPT,), jnp.int32),
        pltpu.VMEM((NH,), jnp.float32),
        pltpu.VMEM((TPS, THR), jnp.float32),
        pltpu.VMEM((THR,), jnp.float32),
        pltpu.VMEM_SHARED((TPS, TPS, THR), jnp.float32),
    ],
    compiler_params=_cp,
)
def _sc_counts(src_hbm, dst_hbm, out_src, out_dst, idx_v, hist_v, red_v, res_v,
               stage_sh):
    c = lax.axis_index("c")
    t = lax.axis_index("s")

    @pl.loop(0, NH // 16)
    def _(i):
        hist_v[pl.ds(i * 16, 16)] = jnp.zeros((16,), jnp.float32)

    @pl.when(c == 0)
    def _():
        pltpu.sync_copy(src_hbm.at[pl.ds(t * EPT, EPT)], idx_v)

    @pl.when(c == 1)
    def _():
        pltpu.sync_copy(dst_hbm.at[pl.ds(t * EPT, EPT)], idx_v)

    ones16 = jnp.full((16,), 1.0, jnp.float32)

    @pl.loop(0, EPT // 16)
    def _(i):
        iv = idx_v[pl.ds(i * 16, 16)]
        plsc.addupdate_scatter(hist_v, [iv], ones16)

    for o in range(TPS):
        pltpu.sync_copy(hist_v.at[pl.ds(o * THR, THR)], stage_sh.at[o, t])
    plsc.subcore_barrier()
    pltpu.sync_copy(stage_sh.at[t], red_v)

    for s0 in range(0, THR, 16):
        acc16 = jnp.zeros((16,), jnp.float32)
        for r in range(TPS):
            acc16 = acc16 + red_v[r, pl.ds(s0, 16)]
        res_v[pl.ds(s0, 16)] = acc16

    @pl.when(c == 0)
    def _():
        pltpu.sync_copy(res_v, out_src.at[pl.ds(t * THR, THR)])

    @pl.when(c == 1)
    def _():
        pltpu.sync_copy(res_v, out_dst.at[pl.ds(t * THR, THR)])


def _zero_rows(buf, nrows):
    @pl.loop(0, nrows)
    def _(r):
        for k in range(8):
            buf[r, pl.ds(k * 16, 16)] = jnp.zeros((16,), jnp.float32)


def _zero_acc(acc_sh, zrow, t, nrows, sem):
    chunk = (nrows // 8) * 8
    base = t * TROWS
    nfull = TROWS // chunk
    hs = [pltpu.async_copy(zrow.at[pl.ds(0, chunk)],
                           acc_sh.at[pl.ds(base + z * chunk, chunk)], sem)
          for z in range(nfull)]
    rem = TROWS - nfull * chunk
    if rem:
        hs.append(pltpu.async_copy(zrow.at[pl.ds(0, rem)],
                                   acc_sh.at[pl.ds(base + nfull * chunk, rem)],
                                   sem))
    for h in hs:
        h.wait()


def _make_seg_streamer(src2d_hbm, dst2d_hbm, acc_sh, bufs, tile_base,
                       seg, g, nseg):
    sidx_a, didx_a, sidx_b, didx_b, sidx_c, didx_c = bufs[:6]
    rows = bufs[6:6 + g]
    isems = bufs[6 + g:10 + g]
    gsems = bufs[10 + g:10 + 2 * g]
    ssems = bufs[10 + 2 * g:10 + 3 * g]

    def seg_load(s, si, di, sems):
        src = src2d_hbm.at[pl.ds(tile_base + s * seg, seg)]
        dst = dst2d_hbm.at[pl.ds(tile_base + s * seg, seg)]
        if sems is None:
            pltpu.sync_copy(src, si)
            pltpu.sync_copy(dst, di)
            return None
        return (pltpu.async_copy(src, si, sems[0]),
                pltpu.async_copy(dst, di, sems[1]))

    def run(x_hbm):
        def sweep(segments, hooks=None):
            total = seg * len(segments)
            gh = [None] * total
            sh = [None] * total
            late = []

            def fire_s(k):
                di = segments[k // seg][1]
                gh[k].wait()
                sh[k] = pltpu.async_copy(rows[k % g], acc_sh.at[di.at[k % seg]],
                                         ssems[k % g], add=True)

            for j in range(total):
                if hooks and j in hooks:
                    late.extend(hooks[j]() or ())
                if j >= g:
                    sh[j - g].wait()
                si = segments[j // seg][0]
                gh[j] = pltpu.async_copy(x_hbm.at[si.at[j % seg]], rows[j % g],
                                         gsems[j % g])
                if j - (g - 1) >= 0:
                    fire_s(j - (g - 1))
            if hooks and total in hooks:
                late.extend(hooks[total]() or ())
            for k in range(max(total - g + 1, 0), total):
                fire_s(k)
            for k in range(max(total - g, 0), total):
                sh[k].wait()
            for h in late:
                h.wait()

        seg_load(0, sidx_a, didx_a, None)

        @pl.loop(0, nseg - 3, step=3)
        def _(s):
            hb = seg_load(s + 1, sidx_b, didx_b, isems[0:2])
            hc = seg_load(s + 2, sidx_c, didx_c, isems[2:4])

            def wait_hb():
                for h in hb:
                    h.wait()

            def wait_hc():
                for h in hc:
                    h.wait()

            def fire_ha():
                return seg_load(s + 3, sidx_a, didx_a, isems[0:2])

            hooks = {seg: wait_hb, 2 * seg: wait_hc}
            pos = min(seg + g, 3 * seg)
            if pos in hooks:
                prev = hooks[pos]

                def merged(prev=prev):
                    prev()
                    return fire_ha()

                hooks[pos] = merged
            else:
                hooks[pos] = fire_ha
            sweep([(sidx_a, didx_a), (sidx_b, didx_b), (sidx_c, didx_c)],
                  hooks)

        sweep([(sidx_a, didx_a)])

    return run


def _prop_scratch(ch, seg, g):
    return ([pltpu.VMEM((seg, ch), jnp.int32) for _ in range(6)]
            + [pltpu.VMEM((ch, 128), jnp.float32) for _ in range(g)]
            + [pltpu.SemaphoreType.DMA for _ in range(4 + 2 * g)])


CH2 = 25
NCH2 = EPT // CH2
SEG2 = 16
NSEG2 = NCH2 // SEG2
G2 = 9


@functools.partial(
    pl.kernel,
    out_type=jax.ShapeDtypeStruct((2, NP, 128), jnp.float32),
    mesh=_mesh,
    scratch_types=[pltpu.VMEM_SHARED((NP, 128), jnp.float32)]
                  + _prop_scratch(CH2, SEG2, G2),
)
def _prop2(x_lo, x_hi, src2d_hbm, dst2d_hbm, out, acc_sh, *bufs):
    c = lax.axis_index("c")
    t = lax.axis_index("s")

    _zero_rows(bufs[6], CH2)
    _zero_acc(acc_sh, bufs[6], t, CH2, bufs[6 + G2])
    plsc.subcore_barrier()

    run = _make_seg_streamer(src2d_hbm, dst2d_hbm, acc_sh, bufs, t * NCH2,
                             SEG2, G2, NSEG2)

    @pl.when(c == 0)
    def _():
        run(x_lo)

    @pl.when(c == 1)
    def _():
        run(x_hi)

    plsc.subcore_barrier()

    @pl.when(c == 0)
    def _():
        pltpu.sync_copy(acc_sh.at[pl.ds(t * TROWS, TROWS)],
                        out.at[0, pl.ds(t * TROWS, TROWS)])

    @pl.when(c == 1)
    def _():
        pltpu.sync_copy(acc_sh.at[pl.ds(t * TROWS, TROWS)],
                        out.at[1, pl.ds(t * TROWS, TROWS)])


SEG3 = 8
NSEG3 = NCH3 // SEG3
G3 = 8


@functools.partial(
    pl.kernel,
    out_type=jax.ShapeDtypeStruct((2, NP, 128), jnp.float32),
    mesh=_mesh,
    scratch_types=[pltpu.VMEM_SHARED((NP, 128), jnp.float32)]
                  + _prop_scratch(CH3, SEG3, G3),
)
def _prop3(xp, src2d_hbm, dst2d_hbm, out, acc_sh, *bufs):
    c = lax.axis_index("c")
    t = lax.axis_index("s")
    half_rows = (E // 2) // CH3

    _zero_rows(bufs[6], CH3)
    _zero_acc(acc_sh, bufs[6], t, CH3, bufs[6 + G3])
    plsc.subcore_barrier()

    run = _make_seg_streamer(src2d_hbm, dst2d_hbm, acc_sh, bufs,
                             c * half_rows + t * NCH3, SEG3, G3, NSEG3)
    run(xp)

    plsc.subcore_barrier()

    @pl.when(c == 0)
    def _():
        pltpu.sync_copy(acc_sh.at[pl.ds(t * TROWS, TROWS)],
                        out.at[0, pl.ds(t * TROWS, TROWS)])

    @pl.when(c == 1)
    def _():
        pltpu.sync_copy(acc_sh.at[pl.ds(t * TROWS, TROWS)],
                        out.at[1, pl.ds(t * TROWS, TROWS)])


RB = 1264


def _row_spec(w):
    return pl.BlockSpec((RB, w), lambda i: (i, 0))


def _full_spec(h, w):
    return pl.BlockSpec((h, w), lambda i: (0, 0))


def _prep_body(cs_ref, cd_ref, x_ref, ns_ref, nd_ref, lo_ref, hi_ref):
    ns = lax.rsqrt(jnp.maximum(cs_ref[...], 1.0))
    nd = lax.rsqrt(jnp.maximum(cd_ref[...], 1.0))
    ns_ref[...] = ns
    nd_ref[...] = nd
    xs = x_ref[...] * ns
    lo_ref[...] = xs[:, :128]
    hi_ref[...] = xs[:, 128:]


def _tc_prep(cs, cd, x):
    return pl.pallas_call(
        _prep_body,
        grid=(NP // RB,),
        in_specs=[_row_spec(1), _row_spec(1), _row_spec(256)],
        out_specs=[_row_spec(1), _row_spec(1), _row_spec(128), _row_spec(128)],
        out_shape=[
            jax.ShapeDtypeStruct((NP, 1), jnp.float32),
            jax.ShapeDtypeStruct((NP, 1), jnp.float32),
            jax.ShapeDtypeStruct((NP, 128), jnp.float32),
            jax.ShapeDtypeStruct((NP, 128), jnp.float32),
        ],
    )(cs, cd, x)


_stk_spec = pl.BlockSpec((2, RB, 128), lambda i: (0, i, 0))


def _l12_body(a_ref, nd_ref, ns_ref, w1_ref, b1_ref, w2_ref,
              glo_ref, ghi_ref):
    agg = jnp.concatenate([a_ref[0], a_ref[1]], axis=1)
    y1 = jnp.dot(agg * nd_ref[...], w1_ref[...],
                 preferred_element_type=jnp.float32)
    y1 = y1 + b1_ref[...]
    m = jnp.max(y1, axis=-1, keepdims=True)
    e = jnp.exp(y1 - m)
    h1 = e / jnp.sum(e, axis=-1, keepdims=True)
    g2 = jnp.dot(h1, w2_ref[...], preferred_element_type=jnp.float32)
    g2 = g2 * ns_ref[...]
    glo_ref[...] = g2[:, :128]
    ghi_ref[...] = g2[:, 128:]


def _tc_l12(a, nd, ns, W1, b1, W2):
    return pl.pallas_call(
        _l12_body,
        grid=(NP // RB,),
        in_specs=[_stk_spec, _row_spec(1), _row_spec(1),
                  _full_spec(256, 512), _full_spec(1, 512),
                  _full_spec(512, 256)],
        out_specs=[_row_spec(128), _row_spec(128)],
        out_shape=[
            jax.ShapeDtypeStruct((NP, 128), jnp.float32),
            jax.ShapeDtypeStruct((NP, 128), jnp.float32),
        ],
    )(a, nd, ns, W1, b1, W2)


def _l23_body(a_ref, nd_ref, ns_ref, b2_ref, w3_ref, gp_ref):
    agg = jnp.concatenate([a_ref[0], a_ref[1]], axis=1)
    y2 = agg * nd_ref[...] + b2_ref[...]
    h2 = jnp.maximum(y2, 0.0)
    g3 = jnp.dot(h2, w3_ref[...], preferred_element_type=jnp.float32)
    g3 = g3 * ns_ref[...]
    gp_ref[...] = jnp.concatenate([g3, jnp.zeros_like(g3)], axis=1)


def _tc_l23(a, nd, ns, b2, W3):
    return pl.pallas_call(
        _l23_body,
        grid=(NP // RB,),
        in_specs=[_stk_spec, _row_spec(1), _row_spec(1),
                  _full_spec(1, 256), _full_spec(256, 64)],
        out_specs=_row_spec(128),
        out_shape=jax.ShapeDtypeStruct((NP, 128), jnp.float32),
    )(a, nd, ns, b2, W3)


def _final_body(p_ref, nd_ref, b3_ref, out_ref):
    agg = p_ref[0][:, :64] + p_ref[1][:, :64]
    out_ref[...] = agg * nd_ref[...] + b3_ref[...]


def _tc_final(p, nd, b3):
    return pl.pallas_call(
        _final_body,
        grid=(NP // RB,),
        in_specs=[pl.BlockSpec((2, RB, 128), lambda i: (0, i, 0)),
                  _row_spec(1), _full_spec(1, 64)],
        out_specs=_row_spec(64),
        out_shape=jax.ShapeDtypeStruct((N, 64), jnp.float32),
    )(p, nd, b3)


def kernel(in_feat, edge_index, W1, b1, W2, b2, W3, b3):
    src2d3 = edge_index[0].reshape(E // CH3, CH3)
    dst2d3 = edge_index[1].reshape(E // CH3, CH3)
    c_src, c_dst = _sc_counts(edge_index[0], edge_index[1])
    ns, nd, xs_lo, xs_hi = _tc_prep(c_src[:NP].reshape(NP, 1),
                                    c_dst[:NP].reshape(NP, 1), in_feat)
    a1 = _prop2(xs_lo, xs_hi, src2d3, dst2d3)
    g_lo, g_hi = _tc_l12(a1, nd, ns, W1, b1.reshape(1, -1), W2)
    a2 = _prop2(g_lo, g_hi, src2d3, dst2d3)
    g3p = _tc_l23(a2, nd, ns, b2.reshape(1, -1), W3)
    p3 = _prop3(g3p, src2d3, dst2d3)
    return _tc_final(p3, nd, b3.reshape(1, -1))
